# Initial kernel scaffold; baseline (speedup 1.0000x reference)
import jax
import jax.numpy as jnp
from jax.experimental import pallas as pl


def kernel(spot_emb, user_emb, user_edge_weight, city_edge_weight, station_edge_weight, category_edge_weight, user_spot, user_edge_index, city_edge_index, station_edge_index, category_edge_index):
    raise NotImplementedError("write your pallas kernel here")


# speedup vs baseline: 3.4304x; 3.4304x over previous
import functools

import jax
import jax.numpy as jnp
from jax import lax
from jax.experimental import pallas as pl
from jax.experimental.pallas import tpu as pltpu
from jax.experimental.pallas import tpu_sc as plsc

NU, MS, D = 27094, 42852, 64
H = D // 2
NT = 16
C = 128
NU_PAD = 28672
MS_PAD = 43008
F32 = jnp.float32
I32 = jnp.int32

_mesh = lambda: plsc.VectorSubcoreMesh(core_axis_name="c", subcore_axis_name="s")
_PARAMS = lambda: pltpu.CompilerParams(use_tc_tiling_on_sc=False,
                                       needs_layout_passes=False)


def _pad_to(x, n, val):
    return jnp.pad(x, (0, n - x.shape[0]), constant_values=val)


def _fill(ref, n, value):
    v = jnp.full((16,), value, F32)

    def body(i, _):
        ref[pl.ds(i * 16, 16)] = v
        return 0

    lax.fori_loop(0, n // 16, body, 0)


def _fill2(ref, rows, value):
    v = jnp.full((16,), value, F32)

    def body(r, _):
        ref[r, pl.ds(0, 16)] = v
        ref[r, pl.ds(16, 16)] = v
        return 0

    lax.fori_loop(0, rows, body, 0)


def _rsqrt16(x):
    i = plsc.bitcast(x, I32)
    i = jnp.int32(0x5F3759DF) - (i >> 1)
    y = plsc.bitcast(i, F32)
    for _ in range(3):
        y = y * (jnp.float32(1.5) - jnp.float32(0.5) * x * y * y)
    return y


def _counts_call(u_idx, s_idx, epad):
    per_t = epad // NT
    nch = per_t // C

    @functools.partial(
        pl.kernel,
        mesh=_mesh(),
        out_type=jax.ShapeDtypeStruct((2, MS_PAD), F32),
        scratch_types=[
            pltpu.VMEM_SHARED((MS_PAD,), F32),
            pltpu.VMEM((C,), I32),
            pltpu.VMEM((C,), F32),
            pltpu.VMEM((MS_PAD // NT,), F32),
        ],
    )
    def k(u_hbm, s_hbm, cnt_out, acc, idx_v, ones_v, z_v):
        c = lax.axis_index("c")
        t = lax.axis_index("s")
        stripe = MS_PAD // NT
        _fill(ones_v, C, 1.0)
        _fill(z_v, stripe, 0.0)
        pltpu.sync_copy(z_v, acc.at[pl.ds(t * stripe, stripe)])
        plsc.subcore_barrier()

        def scatter_from(row_hbm):
            def body(j, _):
                base = t * per_t + j * C
                pltpu.sync_copy(row_hbm.at[pl.ds(base, C)], idx_v)
                pltpu.sync_copy(ones_v, acc.at[idx_v], add=True)
                return 0

            lax.fori_loop(0, nch, body, 0)

        @pl.when(c == 0)
        def _():
            scatter_from(u_hbm)

        @pl.when(c == 1)
        def _():
            scatter_from(s_hbm)

        plsc.subcore_barrier()
        pltpu.sync_copy(acc.at[pl.ds(t * stripe, stripe)],
                        cnt_out.at[c, pl.ds(t * stripe, stripe)])

    return k(u_idx, s_idx)


def _div_call(cnt, u_idx, s_idx, epad):
    per_w = epad // (2 * NT)
    nch = per_w // C

    @functools.partial(
        pl.kernel,
        mesh=_mesh(),
        out_type=jax.ShapeDtypeStruct((epad,), F32),
        scratch_types=[
            pltpu.VMEM((NU_PAD,), F32),
            pltpu.VMEM((MS_PAD,), F32),
            pltpu.VMEM((C,), I32),
            pltpu.VMEM((C,), I32),
            pltpu.VMEM((C,), F32),
        ],
        compiler_params=_PARAMS(),
    )
    def k(cnt_hbm, u_hbm, s_hbm, div_out, uc_v, sc_v, ui_v, si_v, dv_v):
        c = lax.axis_index("c")
        t = lax.axis_index("s")
        wid = t * 2 + c
        pltpu.sync_copy(cnt_hbm.at[0, pl.ds(0, NU_PAD)], uc_v)
        pltpu.sync_copy(cnt_hbm.at[1, pl.ds(0, MS_PAD)], sc_v)

        def body(j, _):
            base = wid * per_w + j * C
            pltpu.sync_copy(u_hbm.at[pl.ds(base, C)], ui_v)
            pltpu.sync_copy(s_hbm.at[pl.ds(base, C)], si_v)
            for kk in range(C // 16):
                u16 = ui_v[pl.ds(kk * 16, 16)]
                s16 = si_v[pl.ds(kk * 16, 16)]
                cu = plsc.load_gather(uc_v, [u16])
                cs = plsc.load_gather(sc_v, [s16])
                dv_v[pl.ds(kk * 16, 16)] = _rsqrt16(cu * cs)
            pltpu.sync_copy(dv_v, div_out.at[pl.ds(base, C)])
            return 0

        lax.fori_loop(0, nch, body, 0)

    return k(cnt, u_idx, s_idx)


def _seg_call(src, dst, w, xsrc, extra, epad, ndst_pad, scale, layer_mode,
              final_scale):
    per_t = epad // NT
    nch = per_t // C
    stripe = ndst_pad // NT
    nz = stripe // C

    outs = jax.ShapeDtypeStruct((2, ndst_pad, H), F32)
    if layer_mode:
        outs = (outs, jax.ShapeDtypeStruct((2, ndst_pad, H), F32))

    def body(src_h, dst_h, w_h, x_h, e_h, *rest):
        if layer_mode:
            xout, oout, acc, srcv, dstv, wv, rows, zr, av, bv, sem = rest
        else:
            xout, acc, srcv, dstv, wv, rows, zr, av, bv, sem = rest
            oout = None
        c = lax.axis_index("c")
        t = lax.axis_index("s")
        _fill2(zr, C, 0.0)

        def zbody(j, _):
            pltpu.sync_copy(zr, acc.at[pl.ds(t * stripe + j * C, C)])
            return 0

        lax.fori_loop(0, nz, zbody, 0)
        plsc.subcore_barrier()

        def ebody(j, _):
            base = t * per_t + j * C
            pltpu.sync_copy(src_h.at[pl.ds(base, C)], srcv)
            pltpu.sync_copy(dst_h.at[pl.ds(base, C)], dstv)
            pltpu.sync_copy(w_h.at[pl.ds(base, C)], wv)
            pltpu.async_copy(x_h.at[c].at[srcv], rows, sem).wait()

            def mbody(g, _):
                w16 = wv[pl.ds(g * 16, 16)]
                for l in range(16):
                    e = g * 16 + l
                    wb = jnp.full((16,), w16[l], F32)
                    rows[e, pl.ds(0, 16)] = rows[e, pl.ds(0, 16)] * wb
                    rows[e, pl.ds(16, 16)] = rows[e, pl.ds(16, 16)] * wb
                return 0

            lax.fori_loop(0, C // 16, mbody, 0)
            pltpu.sync_copy(rows, acc.at[dstv], add=True)
            return 0

        lax.fori_loop(0, nch, ebody, 0)
        plsc.subcore_barrier()

        sc = jnp.float32(scale)
        fs = jnp.float32(final_scale)

        def dbody(j, _):
            r0 = t * stripe + j * C
            pltpu.sync_copy(acc.at[pl.ds(r0, C)], av)
            pltpu.sync_copy(e_h.at[c, pl.ds(r0, C)], bv)

            def rbody(r, _):
                for off in (0, 16):
                    a = av[r, pl.ds(off, 16)]
                    b = bv[r, pl.ds(off, 16)]
                    if layer_mode:
                        x = a * sc
                        av[r, pl.ds(off, 16)] = x
                        bv[r, pl.ds(off, 16)] = (b + x) * fs
                    else:
                        av[r, pl.ds(off, 16)] = b + a * sc
                return 0

            lax.fori_loop(0, C, rbody, 0)
            pltpu.sync_copy(av, xout.at[c, pl.ds(r0, C)])
            if layer_mode:
                pltpu.sync_copy(bv, oout.at[c, pl.ds(r0, C)])
            return 0

        lax.fori_loop(0, nz, dbody, 0)

    scratch = [
        pltpu.VMEM_SHARED((ndst_pad, H), F32),
        pltpu.VMEM((C,), I32),
        pltpu.VMEM((C,), I32),
        pltpu.VMEM((C,), F32),
        pltpu.VMEM((C, H), F32),
        pltpu.VMEM((C, H), F32),
        pltpu.VMEM((C, H), F32),
        pltpu.VMEM((C, H), F32),
        pltpu.SemaphoreType.DMA,
    ]
    k = functools.partial(
        pl.kernel, mesh=_mesh(), out_type=outs, scratch_types=scratch,
        compiler_params=_PARAMS(),
    )(body)
    return k(src, dst, w, xsrc, extra)


def kernel(spot_emb, user_emb, user_edge_weight, city_edge_weight,
           station_edge_weight, category_edge_weight, user_spot,
           user_edge_index, city_edge_index, station_edge_index,
           category_edge_index):
    def split_pad(emb, npad):
        p = jnp.stack([emb[:, :H], emb[:, H:]], axis=0)
        return jnp.pad(p, ((0, 0), (0, npad - emb.shape[0]), (0, 0)))

    spot_p = split_pad(spot_emb, MS_PAD)
    user_p = split_pad(user_emb, NU_PAD)

    e_us = user_spot.shape[1]
    EUS = ((e_us + 4095) // 4096) * 4096
    u_idx = _pad_to(user_spot[0].astype(I32), EUS, NU)
    s_idx = _pad_to(user_spot[1].astype(I32), EUS, MS)

    e_u = user_edge_index.shape[1]
    EU = ((e_u + 2047) // 2048) * 2048
    usrc = _pad_to(user_edge_index[0].astype(I32), EU, 0)
    udst = _pad_to(user_edge_index[1].astype(I32), EU, 0)
    uw = _pad_to(user_edge_weight, EU, 0.0)

    s3src = jnp.concatenate([category_edge_index[0], city_edge_index[0],
                             station_edge_index[0]]).astype(I32)
    s3dst = jnp.concatenate([category_edge_index[1], city_edge_index[1],
                             station_edge_index[1]]).astype(I32)
    s3w = jnp.concatenate([category_edge_weight, city_edge_weight,
                           station_edge_weight])
    ES3 = ((s3src.shape[0] + 2047) // 2048) * 2048
    s3src = _pad_to(s3src, ES3, 0)
    s3dst = _pad_to(s3dst, ES3, 0)
    s3w = _pad_to(s3w, ES3, 0.0)

    cnt = _counts_call(u_idx, s_idx, EUS)
    div = _div_call(cnt, u_idx, s_idx, EUS)

    spot_x = _seg_call(s3src, s3dst, s3w, spot_p, spot_p, ES3, MS_PAD,
                       1.0 / 3.0, False, 1.0)
    user_x = _seg_call(usrc, udst, uw, user_p, user_p, EU, NU_PAD,
                       1.0, False, 1.0)
    spot_o, user_o = spot_x, user_x

    for layer in range(3):
        fs = 0.25 if layer == 2 else 1.0
        user_next, user_o = _seg_call(s_idx, u_idx, div, spot_x, user_o,
                                      EUS, NU_PAD, 1.0, True, fs)
        spot_next, spot_o = _seg_call(u_idx, s_idx, div, user_x, spot_o,
                                      EUS, MS_PAD, 1.0, True, fs)
        spot_x, user_x = spot_next, user_next

    spot_out = jnp.concatenate([spot_o[0, :MS], spot_o[1, :MS]], axis=1)
    user_out = jnp.concatenate([user_o[0, :NU], user_o[1, :NU]], axis=1)
    return (spot_out, user_out)
